# baseline (speedup 1.0000x reference)
import functools

import jax
import jax.numpy as jnp
from jax import lax
from jax.experimental import pallas as pl
from jax.experimental.pallas import tpu as pltpu
from jax.experimental.pallas import tpu_sc as plsc

NS = 16
C = 128
G = 16


def _prop_kernel(N_FULL, D, NB, K):
    RPT = N_FULL // NS
    nz, rem = RPT // C, RPT % C
    mesh = plsc.VectorSubcoreMesh(
        core_axis_name="c", subcore_axis_name="s", num_cores=1)

    @functools.partial(
        pl.kernel,
        out_type=jax.ShapeDtypeStruct((K + 1, N_FULL, D), jnp.float32),
        mesh=mesh,
        scratch_types=[
            pltpu.VMEM((2, G, C), jnp.int32),
            pltpu.VMEM((2, G, C), jnp.int32),
            pltpu.VMEM((2, C, D), jnp.float32),
            pltpu.VMEM_SHARED((N_FULL, D), jnp.float32),
            pltpu.SemaphoreType.DMA,
            pltpu.SemaphoreType.DMA,
            pltpu.SemaphoreType.DMA,
            pltpu.SemaphoreType.DMA,
        ],
    )
    def prop(x_hbm, src_hbm, dst_hbm, zeros_hbm, out_hbm,
             src_blk, dst_blk, rows_v, acc, g0, g1, i0, i1):
        s = lax.axis_index("s")
        base = s * RPT
        gsem = (g0, g1)
        isem = (i0, i1)

        def iprefetch(ib, b):
            pltpu.async_copy(src_hbm.at[s].at[ib], src_blk.at[b], isem[b])
            pltpu.async_copy(dst_hbm.at[s].at[ib], dst_blk.at[b], isem[b])

        def iwait(b):
            pltpu.make_async_copy(
                src_hbm.at[s].at[0], src_blk.at[b], isem[b]).wait()
            pltpu.make_async_copy(
                dst_hbm.at[s].at[0], dst_blk.at[b], isem[b]).wait()

        for z in range(nz):
            pltpu.sync_copy(x_hbm.at[pl.ds(base + z * C, C)],
                            out_hbm.at[0].at[pl.ds(base + z * C, C)])
        if rem:
            pltpu.sync_copy(x_hbm.at[pl.ds(base + nz * C, rem)],
                            out_hbm.at[0].at[pl.ds(base + nz * C, rem)])

        def hop(k, carry):
            for z in range(nz):
                pltpu.sync_copy(zeros_hbm, acc.at[pl.ds(base + z * C, C)])
            if rem:
                pltpu.sync_copy(zeros_hbm.at[pl.ds(0, rem)],
                                acc.at[pl.ds(base + nz * C, rem)])
            plsc.subcore_barrier()

            h_ref = out_hbm.at[k]

            def gather(b, g, rb):
                pltpu.async_copy(h_ref.at[src_blk.at[b].at[g]],
                                 rows_v.at[rb], gsem[rb])

            def gwait(b, rb):
                pltpu.make_async_copy(h_ref.at[src_blk.at[b].at[0]],
                                      rows_v.at[rb], gsem[rb]).wait()

            def scatter(b, g, rb):
                pltpu.sync_copy(rows_v.at[rb],
                                acc.at[dst_blk.at[b].at[g]], add=True)

            def do_block(ib, b):
                iwait(b)
                gather(b, 0, 0)
                for g in range(G):
                    if g + 1 < G:
                        gather(b, g + 1, (g + 1) % 2)
                    gwait(b, g % 2)

                @pl.when(ib + 2 < NB)
                def _():
                    iprefetch(ib + 2, b)

            iprefetch(0, 0)
            iprefetch(1, 1)

            def blockpair(p, c2):
                ib = 2 * p
                do_block(ib, 0)
                do_block(ib + 1, 1)
                return c2
            lax.fori_loop(0, NB // 2, blockpair, 0)
            plsc.subcore_barrier()

            for z in range(nz):
                pltpu.sync_copy(acc.at[pl.ds(base + z * C, C)],
                                out_hbm.at[k + 1].at[pl.ds(base + z * C, C)])
            if rem:
                pltpu.sync_copy(acc.at[pl.ds(base + nz * C, rem)],
                                out_hbm.at[k + 1].at[pl.ds(base + nz * C, rem)])
            return carry
        for k in range(K):
            hop(k, 0)

    return prop


def _att_sum_kernel(hs_ref, att_ref, out_ref):
    acc = att_ref[0] * hs_ref[0]
    for k in range(1, hs_ref.shape[0]):
        acc = acc + att_ref[k] * hs_ref[k]
    out_ref[...] = acc


def kernel(x, edge_index, att):
    N, D = x.shape
    E = edge_index.shape[1]
    K = att.shape[0] - 1

    N_FULL = ((N + C) // C) * C
    blk = 2 * G * C
    per_w = ((E + NS * blk - 1) // (NS * blk)) * blk
    E_pad = per_w * NS
    NB = per_w // (G * C)

    src = jnp.concatenate(
        [edge_index[0], jnp.zeros((E_pad - E,), jnp.int32)]
    ).reshape(NS, NB, G, C)
    dst = jnp.concatenate(
        [edge_index[1], jnp.full((E_pad - E,), N, jnp.int32)]
    ).reshape(NS, NB, G, C)

    x_full = jnp.pad(x, ((0, N_FULL - N), (0, 0)))
    zeros = jnp.zeros((C, D), jnp.float32)

    hs = _prop_kernel(N_FULL, D, NB, K)(x_full, src, dst, zeros)

    BR = 32
    out_full = pl.pallas_call(
        _att_sum_kernel,
        grid=(N_FULL // BR,),
        in_specs=[
            pl.BlockSpec((K + 1, BR, D), lambda i: (0, i, 0)),
            pl.BlockSpec(memory_space=pltpu.SMEM),
        ],
        out_specs=pl.BlockSpec((BR, D), lambda i: (i, 0)),
        out_shape=jax.ShapeDtypeStruct((N_FULL, D), jnp.float32),
    )(hs, att)
    return out_full[:N]


# speedup vs baseline: 1.0955x; 1.0955x over previous
import functools

import jax
import jax.numpy as jnp
from jax import lax
from jax.experimental import pallas as pl
from jax.experimental.pallas import tpu as pltpu
from jax.experimental.pallas import tpu_sc as plsc

NC = 2
NS = 16
NW = NC * NS
C = 128
G = 8


def _prop_kernel(N_FULL, D, NB, K):
    RPT = N_FULL // NS
    RPW = N_FULL // NW
    mesh = plsc.VectorSubcoreMesh(
        core_axis_name="c", subcore_axis_name="s", num_cores=NC)

    @functools.partial(
        pl.kernel,
        out_type=(jax.ShapeDtypeStruct((K + 1, N_FULL, D), jnp.float32),
                  jax.ShapeDtypeStruct((NC, N_FULL, D), jnp.float32)),
        mesh=mesh,
        scratch_types=[
            pltpu.VMEM((2, G, C), jnp.int32),
            pltpu.VMEM((2, G, C), jnp.int32),
            pltpu.VMEM((2, C, D), jnp.float32),
            pltpu.VMEM_SHARED((N_FULL, D), jnp.float32),
            pltpu.SemaphoreType.DMA,
            pltpu.SemaphoreType.DMA,
            pltpu.SemaphoreType.DMA,
            pltpu.SemaphoreType.DMA,
            pltpu.SemaphoreType.REGULAR,
        ],
    )
    def prop(x_hbm, src_hbm, dst_hbm, zeros_hbm,
             out_hbm, part_hbm,
             src_blk, dst_blk, rows_v, acc, g0, g1, i0, i1, xsem):
        cid = lax.axis_index("c")
        s = lax.axis_index("s")
        w = cid * NS + s
        gsem = (g0, g1)
        isem = (i0, i1)

        def copy_rows(src_ref, dst_ref, base, nrows):
            full, tail = nrows // C, nrows % C
            for z in range(full):
                pltpu.sync_copy(src_ref.at[pl.ds(base + z * C, C)],
                                dst_ref.at[pl.ds(base + z * C, C)])
            if tail:
                pltpu.sync_copy(src_ref.at[pl.ds(base + full * C, tail)],
                                dst_ref.at[pl.ds(base + full * C, tail)])

        def gbar():
            plsc.subcore_barrier()
            pl.semaphore_signal(xsem, 1, core_index=1 - cid)
            pl.semaphore_wait(xsem, 1)

        def iprefetch(ib, b):
            pltpu.async_copy(src_hbm.at[w].at[ib], src_blk.at[b], isem[b])
            pltpu.async_copy(dst_hbm.at[w].at[ib], dst_blk.at[b], isem[b])

        def iwait(b):
            pltpu.make_async_copy(
                src_hbm.at[w].at[0], src_blk.at[b], isem[b]).wait()
            pltpu.make_async_copy(
                dst_hbm.at[w].at[0], dst_blk.at[b], isem[b]).wait()

        copy_rows(x_hbm, out_hbm.at[0], w * RPW, RPW)

        def hop(k, carry):
            for z in range(RPT // C):
                pltpu.sync_copy(zeros_hbm, acc.at[pl.ds(s * RPT + z * C, C)])
            gbar()

            h_ref = out_hbm.at[k]

            def gather(b, g, rb):
                pltpu.async_copy(h_ref.at[src_blk.at[b].at[g]],
                                 rows_v.at[rb], gsem[rb])

            def gwait(rb):
                pltpu.make_async_copy(h_ref.at[src_blk.at[0].at[0]],
                                      rows_v.at[rb], gsem[rb]).wait()

            def scatter(b, g, rb):
                pltpu.sync_copy(rows_v.at[rb],
                                acc.at[dst_blk.at[b].at[g]], add=True)

            def do_block(ib, b):
                for g in range(G):
                    if g + 1 < G:
                        gather(b, g + 1, (g + 1) % 2)
                    else:
                        @pl.when(ib + 1 < NB)
                        def _():
                            iwait(1 - b)
                            gather(1 - b, 0, 0)
                    gwait(g % 2)
                    scatter(b, g, g % 2)

                @pl.when(ib + 2 < NB)
                def _():
                    iprefetch(ib + 2, b)

            iprefetch(0, 0)
            iprefetch(1, 1)
            iwait(0)
            gather(0, 0, 0)

            def blockpair(p, c2):
                ib = 2 * p
                do_block(ib, 0)
                do_block(ib + 1, 1)
                return c2
            lax.fori_loop(0, NB // 2, blockpair, 0)

            plsc.subcore_barrier()
            copy_rows(acc, part_hbm.at[cid], s * RPT, RPT)
            gbar()

            base = w * RPW
            full, tail = RPW // C, RPW % C
            sizes = [C] * full + ([tail] if tail else [])
            off = 0
            for n in sizes:
                pltpu.sync_copy(part_hbm.at[0].at[pl.ds(base + off, n)],
                                rows_v.at[0].at[pl.ds(0, n)])
                pltpu.sync_copy(part_hbm.at[1].at[pl.ds(base + off, n)],
                                rows_v.at[1].at[pl.ds(0, n)])

                def addrow(r, c3):
                    for col in range(D // 16):
                        a = rows_v[0, r, pl.ds(col * 16, 16)]
                        b2 = rows_v[1, r, pl.ds(col * 16, 16)]
                        rows_v[0, r, pl.ds(col * 16, 16)] = a + b2
                    return c3
                lax.fori_loop(0, n, addrow, 0)
                pltpu.sync_copy(rows_v.at[0].at[pl.ds(0, n)],
                                out_hbm.at[k + 1].at[pl.ds(base + off, n)])
                off += n
            return carry
        lax.fori_loop(0, K, hop, 0)

    return prop


def _att_sum_kernel(hs_ref, att_ref, out_ref):
    acc = att_ref[0] * hs_ref[0]
    for k in range(1, hs_ref.shape[0]):
        acc = acc + att_ref[k] * hs_ref[k]
    out_ref[...] = acc


def kernel(x, edge_index, att):
    N, D = x.shape
    E = edge_index.shape[1]
    K = att.shape[0] - 1

    M = NS * C
    N_FULL = ((N + M) // M) * M
    blk = 2 * G * C
    per_w = ((E + NW * blk - 1) // (NW * blk)) * blk
    E_pad = per_w * NW
    NB = per_w // (G * C)

    src = jnp.concatenate(
        [edge_index[0], jnp.zeros((E_pad - E,), jnp.int32)]
    ).reshape(NW, NB, G, C)
    dst = jnp.concatenate(
        [edge_index[1], jnp.full((E_pad - E,), N, jnp.int32)]
    ).reshape(NW, NB, G, C)

    x_full = jnp.pad(x, ((0, N_FULL - N), (0, 0)))
    zeros = jnp.zeros((C, D), jnp.float32)

    hs, _ = _prop_kernel(N_FULL, D, NB, K)(x_full, src, dst, zeros)

    BR = 32
    out_full = pl.pallas_call(
        _att_sum_kernel,
        grid=(N_FULL // BR,),
        in_specs=[
            pl.BlockSpec((K + 1, BR, D), lambda i: (0, i, 0)),
            pl.BlockSpec(memory_space=pltpu.SMEM),
        ],
        out_specs=pl.BlockSpec((BR, D), lambda i: (i, 0)),
        out_shape=jax.ShapeDtypeStruct((N_FULL, D), jnp.float32),
    )(hs, att)
    return out_full[:N]


# speedup vs baseline: 1.1227x; 1.0248x over previous
import functools

import jax
import jax.numpy as jnp
from jax import lax
from jax.experimental import pallas as pl
from jax.experimental.pallas import tpu as pltpu
from jax.experimental.pallas import tpu_sc as plsc

NS = 16
C = 128


def _prop_kernel(N_FULL, D, NCH, K):
    RPT = N_FULL // NS
    nz, rem = RPT // C, RPT % C
    mesh = plsc.VectorSubcoreMesh(
        core_axis_name="c", subcore_axis_name="s", num_cores=1)

    @functools.partial(
        pl.kernel,
        out_type=jax.ShapeDtypeStruct((K, N_FULL, D), jnp.float32),
        mesh=mesh,
        scratch_types=[
            pltpu.VMEM((2, C), jnp.int32),
            pltpu.VMEM((2, C), jnp.int32),
            pltpu.VMEM((2, C, D), jnp.float32),
            pltpu.VMEM_SHARED((N_FULL, D), jnp.float32),
            pltpu.SemaphoreType.DMA,
            pltpu.SemaphoreType.DMA,
        ],
    )
    def prop(x_hbm, src_hbm, dst_hbm, zeros_hbm, out_hbm,
             src_v, dst_v, rows_v, acc, sem0, sem1):
        s = lax.axis_index("s")
        base = s * RPT
        sems = (sem0, sem1)

        def stage(j, b):
            pltpu.sync_copy(src_hbm.at[s].at[j], src_v.at[b])
            pltpu.sync_copy(dst_hbm.at[s].at[j], dst_v.at[b])

        for k in range(K):
            for z in range(nz):
                pltpu.sync_copy(zeros_hbm, acc.at[pl.ds(base + z * C, C)])
            if rem:
                pltpu.sync_copy(zeros_hbm.at[pl.ds(0, rem)],
                                acc.at[pl.ds(base + nz * C, rem)])
            plsc.subcore_barrier()

            h_ref = x_hbm if k == 0 else out_hbm.at[k - 1]

            def gather(b):
                pltpu.async_copy(h_ref.at[src_v.at[b]], rows_v.at[b], sems[b])

            def gwait(b):
                pltpu.make_async_copy(
                    h_ref.at[src_v.at[b]], rows_v.at[b], sems[b]).wait()

            def scatter(b):
                pltpu.sync_copy(rows_v.at[b], acc.at[dst_v.at[b]], add=True)

            stage(0, 0)
            gather(0)

            def pair(j2, carry):
                j = 2 * j2
                stage(j + 1, 1)
                gather(1)
                gwait(0)
                scatter(0)
                stage(j + 2, 0)
                gather(0)
                gwait(1)
                scatter(1)
                return carry
            lax.fori_loop(0, NCH // 2, pair, 0)
            gwait(0)
            plsc.subcore_barrier()

            for z in range(nz):
                pltpu.sync_copy(acc.at[pl.ds(base + z * C, C)],
                                out_hbm.at[k].at[pl.ds(base + z * C, C)])
            if rem:
                pltpu.sync_copy(acc.at[pl.ds(base + nz * C, rem)],
                                out_hbm.at[k].at[pl.ds(base + nz * C, rem)])

    return prop


def _att_sum_kernel(x_ref, hs_ref, att_ref, out_ref):
    acc = att_ref[0] * x_ref[...]
    for k in range(hs_ref.shape[0]):
        acc = acc + att_ref[k + 1] * hs_ref[k]
    out_ref[...] = acc


def kernel(x, edge_index, att):
    N, D = x.shape
    E = edge_index.shape[1]
    K = att.shape[0] - 1

    N_FULL = ((N + C) // C) * C
    per_w = ((E + NS * 2 * C - 1) // (NS * 2 * C)) * (2 * C)
    E_pad = per_w * NS
    NCH = per_w // C

    src = jnp.concatenate(
        [edge_index[0], jnp.zeros((E_pad - E,), jnp.int32)]).reshape(NS, NCH, C)
    dst = jnp.concatenate(
        [edge_index[1], jnp.full((E_pad - E,), N, jnp.int32)]).reshape(NS, NCH, C)
    src = jnp.concatenate([src, jnp.zeros((NS, 1, C), jnp.int32)], axis=1)
    dst = jnp.concatenate([dst, jnp.full((NS, 1, C), N, jnp.int32)], axis=1)

    x_full = jnp.pad(x, ((0, N_FULL - N), (0, 0)))
    zeros = jnp.zeros((C, D), jnp.float32)

    hs = _prop_kernel(N_FULL, D, NCH, K)(x_full, src, dst, zeros)

    BR = 32
    out_full = pl.pallas_call(
        _att_sum_kernel,
        grid=(N_FULL // BR,),
        in_specs=[
            pl.BlockSpec((BR, D), lambda i: (i, 0)),
            pl.BlockSpec((K, BR, D), lambda i: (0, i, 0)),
            pl.BlockSpec(memory_space=pltpu.SMEM),
        ],
        out_specs=pl.BlockSpec((BR, D), lambda i: (i, 0)),
        out_shape=jax.ShapeDtypeStruct((N_FULL, D), jnp.float32),
    )(x_full, hs, att)
    return out_full[:N]
